# baseline (speedup 1.0000x reference)
import functools

import jax
import jax.numpy as jnp
from jax import lax
from jax.experimental import pallas as pl
from jax.experimental.pallas import tpu as pltpu
from jax.experimental.pallas import tpu_sc as plsc

_L = 16


def _make_sc_kernel(B, S, DF, N, NW, K):
    P = B * N
    PW = P // NW
    n_chunks = PW // K
    plane = S * S

    mesh = plsc.VectorSubcoreMesh(core_axis_name="c", subcore_axis_name="s")
    NC = mesh.num_cores

    @functools.partial(
        pl.kernel,
        out_type=jax.ShapeDtypeStruct((P, DF), jnp.float32),
        mesh=mesh,
        compiler_params=pltpu.CompilerParams(use_tc_tiling_on_sc=False),
        scratch_types=[
            pltpu.VMEM((K,), jnp.float32),
            pltpu.VMEM((K,), jnp.float32),
            pltpu.VMEM((K,), jnp.float32),
            pltpu.VMEM((3, 4 * K), jnp.int32),
            pltpu.VMEM((3, K), jnp.float32),
            pltpu.VMEM((3, 4 * K, DF), jnp.float32),
            pltpu.VMEM((K, DF), jnp.float32),
            pltpu.SemaphoreType.DMA,
        ],
    )
    def sc_kernel(txy, txz, tyz, xs, ys, zs, out, xv, yv, zv, idxs, fr,
                  rows, outb, sem):
        wid = lax.axis_index("s") * NC + lax.axis_index("c")
        base = wid * PW
        boff = (base // N) * plane

        def chunk_body(g, _):
            off = base + g * K
            pltpu.sync_copy(xs.at[pl.ds(off, K)], xv)
            pltpu.sync_copy(ys.at[pl.ds(off, K)], yv)
            pltpu.sync_copy(zs.at[pl.ds(off, K)], zv)

            for i in range(K // _L):
                sl = pl.ds(i * _L, _L)
                cx = (xv[sl] + 1.0) * (0.5 * (S - 1))
                cy = (yv[sl] + 1.0) * (0.5 * (S - 1))
                cz = (zv[sl] + 1.0) * (0.5 * (S - 1))
                hx = cx.astype(jnp.int32)
                hy = cy.astype(jnp.int32)
                hz = cz.astype(jnp.int32)
                fr[0, sl] = cx - hx.astype(jnp.float32)
                fr[1, sl] = cy - hy.astype(jnp.float32)
                fr[2, sl] = cz - hz.astype(jnp.float32)
                bxy = hx * S + hy + boff
                bxz = hx * S + hz + boff
                byz = hy * S + hz + boff
                p = i * _L
                for t, bb in enumerate((bxy, bxz, byz)):
                    idxs[t, pl.ds(0 * K + p, _L)] = bb
                    idxs[t, pl.ds(1 * K + p, _L)] = bb + 1
                    idxs[t, pl.ds(2 * K + p, _L)] = bb + S
                    idxs[t, pl.ds(3 * K + p, _L)] = bb + (S + 1)

            handles = []
            for t, tbl in enumerate((txy, txz, tyz)):
                handles.append(
                    pltpu.async_copy(tbl.at[idxs.at[t]], rows.at[t], sem))
            for h in handles:
                h.wait()

            def grp_body(g2, _):
                p0 = g2 * _L
                gsl = pl.ds(p0, _L)
                fx16 = fr[0, gsl]
                fy16 = fr[1, gsl]
                fz16 = fr[2, gsl]
                for k in range(_L):
                    i = p0 + k
                    fx = fx16[k]
                    fy = fy16[k]
                    fz = fz16[k]
                    for half in range(DF // _L):
                        sl = pl.ds(half * _L, _L)

                        def bil(t, fa, fb):
                            v00 = rows[t, 0 * K + i, sl]
                            v01 = rows[t, 1 * K + i, sl]
                            v10 = rows[t, 2 * K + i, sl]
                            v11 = rows[t, 3 * K + i, sl]
                            r0 = v00 + fb * (v01 - v00)
                            r1 = v10 + fb * (v11 - v10)
                            return r0 + fa * (r1 - r0)

                        rxy = bil(0, fx, fy)
                        rxz = bil(1, fx, fz)
                        ryz = bil(2, fy, fz)
                        outb[i, sl] = rxy * rxz * ryz
                return 0

            lax.fori_loop(0, K // _L, grp_body, 0)
            pltpu.sync_copy(outb, out.at[pl.ds(off, K)])
            return 0

        lax.fori_loop(0, n_chunks, chunk_body, 0)

    return sc_kernel


def kernel(pxy, pxz, pyz, points):
    B, S, _, DF = pxy.shape
    N = points.shape[1]
    NW, K = 32, 256

    txy = pxy.reshape(B * S * S, DF)
    txz = pxz.reshape(B * S * S, DF)
    tyz = pyz.reshape(B * S * S, DF)
    xs = points[:, :, 0].reshape(-1)
    ys = points[:, :, 1].reshape(-1)
    zs = points[:, :, 2].reshape(-1)

    sc = _make_sc_kernel(B, S, DF, N, NW, K)
    out = sc(txy, txz, tyz, xs, ys, zs)
    return out.reshape(B, N, DF)


# speedup vs baseline: 1.1715x; 1.1715x over previous
import functools

import jax
import jax.numpy as jnp
from jax import lax
from jax.experimental import pallas as pl
from jax.experimental.pallas import tpu as pltpu
from jax.experimental.pallas import tpu_sc as plsc

_L = 16


def _make_sc_kernel(B, S, DF, N, NW, K):
    P = B * N
    PW = P // NW
    n_chunks = PW // K
    plane = S * S

    mesh = plsc.VectorSubcoreMesh(core_axis_name="c", subcore_axis_name="s")
    NC = mesh.num_cores

    def buf_set():
        return [
            pltpu.VMEM((3, K), jnp.float32),
            pltpu.VMEM((3, 4 * K), jnp.int32),
            pltpu.VMEM((3, K), jnp.float32),
            pltpu.VMEM((3, 4 * K, DF), jnp.float32),
            pltpu.VMEM((K, DF), jnp.float32),
            pltpu.SemaphoreType.DMA,
        ]

    @functools.partial(
        pl.kernel,
        out_type=jax.ShapeDtypeStruct((P, DF), jnp.float32),
        mesh=mesh,
        compiler_params=pltpu.CompilerParams(use_tc_tiling_on_sc=False),
        scratch_types=buf_set() + buf_set(),
    )
    def sc_kernel(txy, txz, tyz, xs, ys, zs, out,
                  ptsA, idxA, frA, rowsA, outA, semA,
                  ptsB, idxB, frB, rowsB, outB, semB):
        wid = lax.axis_index("s") * NC + lax.axis_index("c")
        base = wid * PW
        boff = (base // N) * plane
        tables = (txy, txz, tyz)

        def fire(g, pts, idxs, fr, rows, sem):
            off = base + g * K
            pltpu.sync_copy(xs.at[pl.ds(off, K)], pts.at[0])
            pltpu.sync_copy(ys.at[pl.ds(off, K)], pts.at[1])
            pltpu.sync_copy(zs.at[pl.ds(off, K)], pts.at[2])
            for i in range(K // _L):
                sl = pl.ds(i * _L, _L)
                cx = (pts[0, sl] + 1.0) * (0.5 * (S - 1))
                cy = (pts[1, sl] + 1.0) * (0.5 * (S - 1))
                cz = (pts[2, sl] + 1.0) * (0.5 * (S - 1))
                hx = cx.astype(jnp.int32)
                hy = cy.astype(jnp.int32)
                hz = cz.astype(jnp.int32)
                fr[0, sl] = cx - hx.astype(jnp.float32)
                fr[1, sl] = cy - hy.astype(jnp.float32)
                fr[2, sl] = cz - hz.astype(jnp.float32)
                bxy = hx * S + hy + boff
                bxz = hx * S + hz + boff
                byz = hy * S + hz + boff
                p = i * _L
                for t, bb in enumerate((bxy, bxz, byz)):
                    idxs[t, pl.ds(0 * K + p, _L)] = bb
                    idxs[t, pl.ds(1 * K + p, _L)] = bb + 1
                    idxs[t, pl.ds(2 * K + p, _L)] = bb + S
                    idxs[t, pl.ds(3 * K + p, _L)] = bb + (S + 1)
            for t, tbl in enumerate(tables):
                pltpu.async_copy(tbl.at[idxs.at[t]], rows.at[t], sem)

        def drain(idxs, rows, sem):
            for t, tbl in enumerate(tables):
                pltpu.make_async_copy(tbl.at[idxs.at[t]], rows.at[t],
                                      sem).wait()

        def combine(g, fr, rows, outb):
            def grp_body(g2, _):
                p0 = g2 * _L
                gsl = pl.ds(p0, _L)
                fx16 = fr[0, gsl]
                fy16 = fr[1, gsl]
                fz16 = fr[2, gsl]
                for k in range(_L):
                    i = p0 + k
                    fx = fx16[k]
                    fy = fy16[k]
                    fz = fz16[k]
                    for half in range(DF // _L):
                        sl = pl.ds(half * _L, _L)

                        def bil(t, fa, fb):
                            v00 = rows[t, 0 * K + i, sl]
                            v01 = rows[t, 1 * K + i, sl]
                            v10 = rows[t, 2 * K + i, sl]
                            v11 = rows[t, 3 * K + i, sl]
                            r0 = v00 + fb * (v01 - v00)
                            r1 = v10 + fb * (v11 - v10)
                            return r0 + fa * (r1 - r0)

                        rxy = bil(0, fx, fy)
                        rxz = bil(1, fx, fz)
                        ryz = bil(2, fy, fz)
                        outb[i, sl] = rxy * rxz * ryz
                return 0

            lax.fori_loop(0, K // _L, grp_body, 0)
            pltpu.sync_copy(outb, out.at[pl.ds(base + g * K, K)])

        fire(0, ptsA, idxA, frA, rowsA, semA)

        def body2(h, _):
            g = h * 2
            fire(g + 1, ptsB, idxB, frB, rowsB, semB)
            drain(idxA, rowsA, semA)
            combine(g, frA, rowsA, outA)

            @pl.when(g + 2 < n_chunks)
            def _():
                fire(g + 2, ptsA, idxA, frA, rowsA, semA)

            drain(idxB, rowsB, semB)
            combine(g + 1, frB, rowsB, outB)
            return 0

        lax.fori_loop(0, n_chunks // 2, body2, 0)

    return sc_kernel


def kernel(pxy, pxz, pyz, points):
    B, S, _, DF = pxy.shape
    N = points.shape[1]
    NW, K = 32, 128

    txy = pxy.reshape(B * S * S, DF)
    txz = pxz.reshape(B * S * S, DF)
    tyz = pyz.reshape(B * S * S, DF)
    xs = points[:, :, 0].reshape(-1)
    ys = points[:, :, 1].reshape(-1)
    zs = points[:, :, 2].reshape(-1)

    sc = _make_sc_kernel(B, S, DF, N, NW, K)
    out = sc(txy, txz, tyz, xs, ys, zs)
    return out.reshape(B, N, DF)
